# baseline (speedup 1.0000x reference)
import functools

import jax
import jax.numpy as jnp
from jax import lax
from jax.experimental import pallas as pl
from jax.experimental.pallas import tpu as pltpu
from jax.experimental.pallas import tpu_sc as plsc

H = 1024
ROW0 = 248
ROWS = 264
COL0 = 128
COLS = 384
NW = 32
ROWS_W = H // NW
CROWS = 8
NCHUNK = ROWS_W // CROWS
VECS = CROWS * H // 16

_mesh = plsc.VectorSubcoreMesh(core_axis_name="c", subcore_axis_name="s")


@functools.partial(
    pl.kernel,
    mesh=_mesh,
    out_type=jax.ShapeDtypeStruct((H, H), jnp.float32),
    scratch_types=[
        pltpu.VMEM((ROWS, COLS), jnp.float32),
        pltpu.VMEM((CROWS, 16, 128), jnp.float32),
        pltpu.VMEM((CROWS, H), jnp.float32),
    ],
    compiler_params=pltpu.CompilerParams(needs_layout_passes=False),
)
def _sample(x_hbm, grid_hbm, out_hbm, tab_v, cin_v, cout_v):
    wid = lax.axis_index("s") * 2 + lax.axis_index("c")
    base_row = wid * ROWS_W
    pltpu.sync_copy(grid_hbm.at[pl.ds(ROW0, ROWS), pl.ds(COL0, COLS)], tab_v)

    def chunk_body(ci, carry):
        crow = base_row + ci * CROWS
        pltpu.sync_copy(x_hbm.at[pl.ds(crow, CROWS)], cin_v)

        @plsc.parallel_loop(0, CROWS * 8, 1, unroll=2)
        def blk_body(j):
            r = j >> 3
            wb = j & 7
            kb = wb * 2
            cb = wb * 128
            for s in range(8):
                wl0 = s * 16
                gx = cin_v[r, kb, pl.ds(wl0, 16)]
                gy = cin_v[r, kb + 1, pl.ds(wl0, 16)]
                ix = gx * 256.0 + 255.5
                iy = gy * 256.0 + 255.5
                xi = ix.astype(jnp.int32)
                yi = iy.astype(jnp.int32)
                fx = ix - xi.astype(jnp.float32)
                fy = iy - yi.astype(jnp.float32)
                cx = xi - COL0
                dy = yi - ROW0
                inx = xi < 511
                iny = yi < 511
                sx = jnp.where(inx, 1, 0)
                sy = jnp.where(iny, 1, 0)
                wx1 = jnp.where(inx, fx, 0.0)
                wy1 = jnp.where(iny, fy, 0.0)
                wx0 = 1.0 - fx
                wy0 = 1.0 - fy
                cx1 = cx + sx
                dy1 = dy + sy
                v00 = plsc.load_gather(tab_v, [dy, cx])
                v01 = plsc.load_gather(tab_v, [dy, cx1])
                v10 = plsc.load_gather(tab_v, [dy1, cx])
                v11 = plsc.load_gather(tab_v, [dy1, cx1])
                res = (v00 * wx0 + v01 * wx1) * wy0 + (v10 * wx0 + v11 * wx1) * wy1
                cout_v[r, pl.ds(cb + wl0, 16)] = res
        pltpu.sync_copy(cout_v, out_hbm.at[pl.ds(crow, CROWS)])
        return carry

    lax.fori_loop(0, NCHUNK, chunk_body, 0)


def kernel(x, grid):
    xv = x.reshape(H, 8, 128, 2).transpose(0, 1, 3, 2).reshape(H, 16, 128)
    g2 = grid.reshape(512, 512)
    out = _sample(xv, g2)
    return out.reshape(1, 1, H, H)


# speedup vs baseline: 1.1664x; 1.1664x over previous
import functools

import jax
import jax.numpy as jnp
from jax import lax
from jax.experimental import pallas as pl
from jax.experimental.pallas import tpu as pltpu
from jax.experimental.pallas import tpu_sc as plsc

H = 1024
ROW0 = 248
ROWS = 264
COL0 = 128
COLS = 384
NW = 32
ROWS_W = H // NW
CROWS = 8
NCHUNK = ROWS_W // CROWS
VECS = CROWS * H // 16

_mesh = plsc.VectorSubcoreMesh(core_axis_name="c", subcore_axis_name="s")


@functools.partial(
    pl.kernel,
    mesh=_mesh,
    out_type=jax.ShapeDtypeStruct((H, H), jnp.float32),
    scratch_types=[
        pltpu.VMEM((ROWS, COLS), jnp.float32),
        pltpu.VMEM((CROWS, 16, 128), jnp.float32),
        pltpu.VMEM((CROWS, H), jnp.float32),
    ],
    compiler_params=pltpu.CompilerParams(needs_layout_passes=False),
)
def _sample(x_hbm, grid_hbm, out_hbm, tab_v, cin_v, cout_v):
    wid = lax.axis_index("s") * 2 + lax.axis_index("c")
    base_row = wid * ROWS_W
    pltpu.sync_copy(grid_hbm.at[pl.ds(ROW0, ROWS), pl.ds(COL0, COLS)], tab_v)

    def chunk_body(ci, carry):
        crow = base_row + ci * CROWS
        pltpu.sync_copy(x_hbm.at[pl.ds(crow, CROWS)], cin_v)

        @plsc.parallel_loop(0, VECS, 1, unroll=16)
        def vec_body(j):
            r = j >> 6
            u = j & 63
            kb = (u >> 3) * 2
            wl0 = (u & 7) * 16
            gx = cin_v[r, kb, pl.ds(wl0, 16)]
            gy = cin_v[r, kb + 1, pl.ds(wl0, 16)]
            ix = gx * 256.0 + 255.5
            iy = gy * 256.0 + 255.5
            xi = ix.astype(jnp.int32)
            yi = iy.astype(jnp.int32)
            fx = ix - xi.astype(jnp.float32)
            fy = iy - yi.astype(jnp.float32)
            cx = xi - COL0
            dy = yi - ROW0
            inx = xi < 511
            iny = yi < 511
            sx = jnp.where(inx, 1, 0)
            sy = jnp.where(iny, 1, 0)
            wx1 = jnp.where(inx, fx, 0.0)
            wy1 = jnp.where(iny, fy, 0.0)
            wx0 = 1.0 - fx
            wy0 = 1.0 - fy
            cx1 = cx + sx
            dy1 = dy + sy
            v00 = plsc.load_gather(tab_v, [dy, cx])
            v01 = plsc.load_gather(tab_v, [dy, cx1])
            v10 = plsc.load_gather(tab_v, [dy1, cx])
            v11 = plsc.load_gather(tab_v, [dy1, cx1])
            res = (v00 * wx0 + v01 * wx1) * wy0 + (v10 * wx0 + v11 * wx1) * wy1
            cout_v[r, pl.ds(u * 16, 16)] = res
        pltpu.sync_copy(cout_v, out_hbm.at[pl.ds(crow, CROWS)])
        return carry

    lax.fori_loop(0, NCHUNK, chunk_body, 0)


def kernel(x, grid):
    xv = x.reshape(H, 8, 128, 2).transpose(0, 1, 3, 2).reshape(H, 16, 128)
    g2 = grid.reshape(512, 512)
    out = _sample(xv, g2)
    return out.reshape(1, 1, H, H)
